# Initial kernel scaffold; baseline (speedup 1.0000x reference)
import jax
import jax.numpy as jnp
from jax.experimental import pallas as pl


def kernel(x, edge_index, W1, b1, W2, b2, W3, b3, W4, b4):
    raise NotImplementedError("write your pallas kernel here")


# speedup vs baseline: 31.1961x; 31.1961x over previous
import functools

import jax
import jax.numpy as jnp
from jax import lax
from jax.experimental import pallas as pl
from jax.experimental.pallas import tpu as pltpu
from jax.experimental.pallas import tpu_sc as plsc

N = 10000
F_IN = 128
H = 16

NC = 2
NS = 16
NW = NC * NS

CW = 128
NBUF = 8
NCH = 80
NG = NCH // NBUF
EPT = NCH * CW
EP = EPT * NW

NP = 10112
RPT = NP // NS

_mesh = plsc.VectorSubcoreMesh(
    core_axis_name="c", subcore_axis_name="s", num_cores=NC, num_subcores=NS
)


@functools.partial(
    pl.kernel,
    out_type=jax.ShapeDtypeStruct((NC, NP, H), jnp.float32),
    mesh=_mesh,
    compiler_params=pltpu.CompilerParams(use_tc_tiling_on_sc=False),
    scratch_types=(
        [
            pltpu.VMEM((NCH, CW), jnp.int32),
            pltpu.VMEM((NCH, CW), jnp.int32),
            pltpu.VMEM_SHARED((NP, H), jnp.float32),
        ]
        + [pltpu.VMEM((CW, H), jnp.float32) for _ in range(NBUF)]
        + [pltpu.SemaphoreType.DMA for _ in range(2 * NBUF)]
    ),
)
def _sc_edge_aggregate(g_hbm, src_hbm, dst_hbm, zeros_hbm, out_hbm,
                       src_v, dst_v, acc, *rest):
    bufs = rest[:NBUF]
    gsems = rest[NBUF:2 * NBUF]
    ssems = rest[2 * NBUF:]
    c = lax.axis_index("c")
    s = lax.axis_index("s")
    w = c * NS + s
    r0 = s * RPT

    pltpu.sync_copy(src_hbm.at[w], src_v)
    pltpu.sync_copy(dst_hbm.at[w], dst_v)
    pltpu.sync_copy(zeros_hbm.at[pl.ds(r0, RPT)], acc.at[pl.ds(r0, RPT)])
    plsc.subcore_barrier()

    for b in range(NBUF):
        pltpu.async_copy(g_hbm.at[src_v.at[b]], bufs[b], gsems[b])

    def group(t, carry):
        for b in range(NBUF):
            j = t * NBUF + b
            pltpu.make_async_copy(g_hbm.at[src_v.at[j]], bufs[b], gsems[b]).wait()
            pltpu.async_copy(bufs[b], acc.at[dst_v.at[j]], ssems[b], add=True)
        for b in range(NBUF):
            j = t * NBUF + b
            pltpu.make_async_copy(bufs[b], acc.at[dst_v.at[j]], ssems[b]).wait()
            jn = j + NBUF

            @pl.when(jn < NCH)
            def _():
                pltpu.async_copy(g_hbm.at[src_v.at[jn]], bufs[b], gsems[b])
        return carry

    lax.fori_loop(0, NG, group, 0)
    plsc.subcore_barrier()
    pltpu.sync_copy(acc.at[pl.ds(r0, RPT)], out_hbm.at[c, pl.ds(r0, RPT)])


@functools.partial(
    pl.kernel,
    out_type=jax.ShapeDtypeStruct((NC, NP, H), jnp.float32),
    mesh=_mesh,
    compiler_params=pltpu.CompilerParams(use_tc_tiling_on_sc=False),
    scratch_types=(
        pltpu.VMEM((NCH, CW), jnp.int32),
        pltpu.VMEM_SHARED((NP, H), jnp.float32),
        pltpu.VMEM((CW, H), jnp.float32),
    ),
)
def _sc_degree(dst_hbm, zeros_hbm, ones_hbm, out_hbm, dst_v, acc, ones_v):
    c = lax.axis_index("c")
    s = lax.axis_index("s")
    w = c * NS + s
    r0 = s * RPT

    pltpu.sync_copy(dst_hbm.at[w], dst_v)
    pltpu.sync_copy(ones_hbm, ones_v)
    pltpu.sync_copy(zeros_hbm.at[pl.ds(r0, RPT)], acc.at[pl.ds(r0, RPT)])
    plsc.subcore_barrier()

    def chunk(j, carry):
        pltpu.sync_copy(ones_v, acc.at[dst_v.at[j]], add=True)
        return carry

    lax.fori_loop(0, NCH, chunk, 0)
    plsc.subcore_barrier()
    pltpu.sync_copy(acc.at[pl.ds(r0, RPT)], out_hbm.at[c, pl.ds(r0, RPT)])


def _tc_prologue(x, W1, dacc):
    def body(x_ref, w_ref, dacc_ref, dinv_ref, g_ref):
        deg = dacc_ref[0] + dacc_ref[1] + 1.0
        dinv = lax.rsqrt(deg)
        dinv_ref[...] = dinv
        h = jnp.dot(x_ref[...], w_ref[...], preferred_element_type=jnp.float32)
        g_ref[...] = dinv * h

    return pl.pallas_call(
        body,
        out_shape=(
            jax.ShapeDtypeStruct((NP, H), jnp.float32),
            jax.ShapeDtypeStruct((NP, H), jnp.float32),
        ),
    )(x, W1, dacc)


def _tc_mid(acc, g, b, W, dinv):
    def body(acc_ref, g_ref, b_ref, w_ref, dinv_ref, out_ref):
        dinv = dinv_ref[...]
        z = dinv * (acc_ref[0] + acc_ref[1] + g_ref[...]) + b_ref[...]
        h = jnp.where(z >= 0.0, z, 0.01 * z)
        out_ref[...] = dinv * jnp.dot(h, w_ref[...],
                                      preferred_element_type=jnp.float32)

    return pl.pallas_call(
        body,
        out_shape=jax.ShapeDtypeStruct((NP, H), jnp.float32),
    )(acc, g, b, W, dinv)


def _tc_final(acc, g, b, dinv):
    def body(acc_ref, g_ref, b_ref, dinv_ref, out_ref):
        z = dinv_ref[...] * (acc_ref[0] + acc_ref[1] + g_ref[...]) + b_ref[...]
        z = z - jnp.max(z, axis=1, keepdims=True)
        e = jnp.exp(z)
        out_ref[...] = e / jnp.sum(e, axis=1, keepdims=True)

    return pl.pallas_call(
        body,
        out_shape=jax.ShapeDtypeStruct((NP, H), jnp.float32),
    )(acc, g, b, dinv)


def kernel(x, edge_index, W1, b1, W2, b2, W3, b3, W4, b4):
    f32 = jnp.float32
    pad_e = jnp.full((EP - edge_index.shape[1],), N, jnp.int32)
    src_r = jnp.concatenate([edge_index[0].astype(jnp.int32), pad_e]).reshape(NW, NCH, CW)
    dst_r = jnp.concatenate([edge_index[1].astype(jnp.int32), pad_e]).reshape(NW, NCH, CW)

    x_pad = jnp.zeros((NP, F_IN), f32).at[:N].set(x)
    zeros = jnp.zeros((NP, H), f32)
    ones_rows = jnp.ones((CW, H), f32)

    dacc = _sc_degree(dst_r, zeros, ones_rows)
    dinv, g = _tc_prologue(x_pad, W1, dacc)

    for b, Wn in ((b1, W2), (b2, W3), (b3, W4)):
        acc = _sc_edge_aggregate(g, src_r, dst_r, zeros)
        g = _tc_mid(acc, g, b.reshape(1, H), Wn, dinv)

    acc = _sc_edge_aggregate(g, src_r, dst_r, zeros)
    out = _tc_final(acc, g, b4.reshape(1, H), dinv)
    return out[:N]


# speedup vs baseline: 47.1466x; 1.5113x over previous
import functools

import jax
import jax.numpy as jnp
from jax import lax
from jax.experimental import pallas as pl
from jax.experimental.pallas import tpu as pltpu
from jax.experimental.pallas import tpu_sc as plsc

N = 10000
F_IN = 128
H = 16

NC = 2
NS = 16
NW = NC * NS

CW = 128
NBUF = 8
NCH = 80
NG = NCH // NBUF
EPT = NCH * CW
EP = EPT * NW

NP = 10112
RPT = NP // NS

_mesh = plsc.VectorSubcoreMesh(
    core_axis_name="c", subcore_axis_name="s", num_cores=NC, num_subcores=NS
)


@functools.partial(
    pl.kernel,
    out_type=jax.ShapeDtypeStruct((NC, NP, H), jnp.float32),
    mesh=_mesh,
    compiler_params=pltpu.CompilerParams(use_tc_tiling_on_sc=False),
    scratch_types=(
        [
            pltpu.VMEM((NCH, CW), jnp.int32),
            pltpu.VMEM((NCH, CW), jnp.int32),
            pltpu.VMEM_SHARED((NP, H), jnp.float32),
            pltpu.VMEM_SHARED((NP, H), jnp.float32),
        ]
        + [pltpu.VMEM((CW, H), jnp.float32) for _ in range(NBUF)]
        + [pltpu.SemaphoreType.DMA for _ in range(2 * NBUF)]
    ),
)
def _sc_edge_aggregate(g_hbm, src_hbm, dst_hbm, zeros_hbm, out_hbm,
                       src_v, dst_v, acc, g_sh, *rest):
    bufs = rest[:NBUF]
    gsems = rest[NBUF:2 * NBUF]
    ssems = rest[2 * NBUF:]
    c = lax.axis_index("c")
    s = lax.axis_index("s")
    w = c * NS + s
    r0 = s * RPT

    pltpu.sync_copy(src_hbm.at[w], src_v)
    pltpu.sync_copy(dst_hbm.at[w], dst_v)
    pltpu.sync_copy(zeros_hbm.at[pl.ds(r0, RPT)], acc.at[pl.ds(r0, RPT)])
    pltpu.sync_copy(g_hbm.at[pl.ds(r0, RPT)], g_sh.at[pl.ds(r0, RPT)])
    plsc.subcore_barrier()

    for b in range(NBUF):
        pltpu.async_copy(g_sh.at[src_v.at[b]], bufs[b], gsems[b])

    def group(t, carry):
        for b in range(NBUF):
            j = t * NBUF + b
            pltpu.make_async_copy(g_sh.at[src_v.at[j]], bufs[b], gsems[b]).wait()
            pltpu.async_copy(bufs[b], acc.at[dst_v.at[j]], ssems[b], add=True)
        for b in range(NBUF):
            j = t * NBUF + b
            pltpu.make_async_copy(bufs[b], acc.at[dst_v.at[j]], ssems[b]).wait()
            jn = j + NBUF

            @pl.when(jn < NCH)
            def _():
                pltpu.async_copy(g_sh.at[src_v.at[jn]], bufs[b], gsems[b])
        return carry

    lax.fori_loop(0, NG, group, 0)
    plsc.subcore_barrier()
    pltpu.sync_copy(acc.at[pl.ds(r0, RPT)], out_hbm.at[c, pl.ds(r0, RPT)])


@functools.partial(
    pl.kernel,
    out_type=jax.ShapeDtypeStruct((NC, NP, H), jnp.float32),
    mesh=_mesh,
    compiler_params=pltpu.CompilerParams(use_tc_tiling_on_sc=False),
    scratch_types=(
        pltpu.VMEM((NCH, CW), jnp.int32),
        pltpu.VMEM_SHARED((NP, H), jnp.float32),
        pltpu.VMEM((CW, H), jnp.float32),
    ),
)
def _sc_degree(dst_hbm, zeros_hbm, ones_hbm, out_hbm, dst_v, acc, ones_v):
    c = lax.axis_index("c")
    s = lax.axis_index("s")
    w = c * NS + s
    r0 = s * RPT

    pltpu.sync_copy(dst_hbm.at[w], dst_v)
    pltpu.sync_copy(ones_hbm, ones_v)
    pltpu.sync_copy(zeros_hbm.at[pl.ds(r0, RPT)], acc.at[pl.ds(r0, RPT)])
    plsc.subcore_barrier()

    def chunk(j, carry):
        pltpu.sync_copy(ones_v, acc.at[dst_v.at[j]], add=True)
        return carry

    lax.fori_loop(0, NCH, chunk, 0)
    plsc.subcore_barrier()
    pltpu.sync_copy(acc.at[pl.ds(r0, RPT)], out_hbm.at[c, pl.ds(r0, RPT)])


def _tc_prologue(x, W1, dacc):
    def body(x_ref, w_ref, dacc_ref, dinv_ref, g_ref):
        deg = dacc_ref[0] + dacc_ref[1] + 1.0
        dinv = lax.rsqrt(deg)
        dinv_ref[...] = dinv
        h = jnp.dot(x_ref[...], w_ref[...], preferred_element_type=jnp.float32)
        g_ref[...] = dinv * h

    return pl.pallas_call(
        body,
        out_shape=(
            jax.ShapeDtypeStruct((NP, H), jnp.float32),
            jax.ShapeDtypeStruct((NP, H), jnp.float32),
        ),
    )(x, W1, dacc)


def _tc_mid(acc, g, b, W, dinv):
    def body(acc_ref, g_ref, b_ref, w_ref, dinv_ref, out_ref):
        dinv = dinv_ref[...]
        z = dinv * (acc_ref[0] + acc_ref[1] + g_ref[...]) + b_ref[...]
        h = jnp.where(z >= 0.0, z, 0.01 * z)
        out_ref[...] = dinv * jnp.dot(h, w_ref[...],
                                      preferred_element_type=jnp.float32)

    return pl.pallas_call(
        body,
        out_shape=jax.ShapeDtypeStruct((NP, H), jnp.float32),
    )(acc, g, b, W, dinv)


def _tc_final(acc, g, b, dinv):
    def body(acc_ref, g_ref, b_ref, dinv_ref, out_ref):
        z = dinv_ref[...] * (acc_ref[0] + acc_ref[1] + g_ref[...]) + b_ref[...]
        z = z - jnp.max(z, axis=1, keepdims=True)
        e = jnp.exp(z)
        out_ref[...] = e / jnp.sum(e, axis=1, keepdims=True)

    return pl.pallas_call(
        body,
        out_shape=jax.ShapeDtypeStruct((NP, H), jnp.float32),
    )(acc, g, b, dinv)


def kernel(x, edge_index, W1, b1, W2, b2, W3, b3, W4, b4):
    f32 = jnp.float32
    pad_e = jnp.full((EP - edge_index.shape[1],), N, jnp.int32)
    src_r = jnp.concatenate([edge_index[0].astype(jnp.int32), pad_e]).reshape(NW, NCH, CW)
    dst_r = jnp.concatenate([edge_index[1].astype(jnp.int32), pad_e]).reshape(NW, NCH, CW)

    x_pad = jnp.zeros((NP, F_IN), f32).at[:N].set(x)
    zeros = jnp.zeros((NP, H), f32)
    ones_rows = jnp.ones((CW, H), f32)

    dacc = _sc_degree(dst_r, zeros, ones_rows)
    dinv, g = _tc_prologue(x_pad, W1, dacc)

    for b, Wn in ((b1, W2), (b2, W3), (b3, W4)):
        acc = _sc_edge_aggregate(g, src_r, dst_r, zeros)
        g = _tc_mid(acc, g, b.reshape(1, H), Wn, dinv)

    acc = _sc_edge_aggregate(g, src_r, dst_r, zeros)
    out = _tc_final(acc, g, b4.reshape(1, H), dinv)
    return out[:N]


# speedup vs baseline: 68.3565x; 1.4499x over previous
import functools

import jax
import jax.numpy as jnp
from jax import lax
from jax.experimental import pallas as pl
from jax.experimental.pallas import tpu as pltpu
from jax.experimental.pallas import tpu_sc as plsc

N = 10000
E = 320000
F_IN = 128
H = 16

NC = 2
NS = 16
NW = NC * NS

CW = 80
NCH = E // NW // CW
NBUF = 5
NG = NCH // NBUF
RPT = N // NS

NPK = N * H // 128

_mesh = plsc.VectorSubcoreMesh(
    core_axis_name="c", subcore_axis_name="s", num_cores=NC, num_subcores=NS
)


@functools.partial(
    pl.kernel,
    out_type=jax.ShapeDtypeStruct((NC, N, H), jnp.float32),
    mesh=_mesh,
    compiler_params=pltpu.CompilerParams(use_tc_tiling_on_sc=False),
    scratch_types=(
        [
            pltpu.VMEM((NCH, CW), jnp.int32),
            pltpu.VMEM((NCH, CW), jnp.int32),
            pltpu.VMEM_SHARED((N, H), jnp.float32),
            pltpu.VMEM_SHARED((N, H), jnp.float32),
        ]
        + [pltpu.VMEM((CW, H), jnp.float32) for _ in range(NBUF)]
        + [pltpu.SemaphoreType.DMA for _ in range(2 * NBUF)]
    ),
)
def _sc_edge_aggregate(g_hbm, src_hbm, dst_hbm, zeros_hbm, out_hbm,
                       src_v, dst_v, acc, g_sh, *rest):
    bufs = rest[:NBUF]
    gsems = rest[NBUF:2 * NBUF]
    ssems = rest[2 * NBUF:]
    c = lax.axis_index("c")
    s = lax.axis_index("s")
    w = c * NS + s
    r0 = s * RPT

    pltpu.sync_copy(src_hbm.at[pl.ds(w * NCH, NCH)], src_v)
    pltpu.sync_copy(dst_hbm.at[pl.ds(w * NCH, NCH)], dst_v)
    pltpu.sync_copy(zeros_hbm.at[pl.ds(r0, RPT)], acc.at[pl.ds(r0, RPT)])
    pltpu.sync_copy(g_hbm.at[pl.ds(r0, RPT)], g_sh.at[pl.ds(r0, RPT)])
    plsc.subcore_barrier()

    for b in range(NBUF):
        pltpu.async_copy(g_sh.at[src_v.at[b]], bufs[b], gsems[b])

    def group(t, carry):
        for b in range(NBUF):
            j = t * NBUF + b
            pltpu.make_async_copy(g_sh.at[src_v.at[j]], bufs[b], gsems[b]).wait()
            pltpu.async_copy(bufs[b], acc.at[dst_v.at[j]], ssems[b], add=True)
        for b in range(NBUF):
            j = t * NBUF + b
            pltpu.make_async_copy(bufs[b], acc.at[dst_v.at[j]], ssems[b]).wait()
            jn = j + NBUF

            @pl.when(jn < NCH)
            def _():
                pltpu.async_copy(g_sh.at[src_v.at[jn]], bufs[b], gsems[b])
        return carry

    lax.fori_loop(0, NG, group, 0)
    plsc.subcore_barrier()
    pltpu.sync_copy(acc.at[pl.ds(r0, RPT)], out_hbm.at[c, pl.ds(r0, RPT)])


@functools.partial(
    pl.kernel,
    out_type=jax.ShapeDtypeStruct((NC, N, H), jnp.float32),
    mesh=_mesh,
    compiler_params=pltpu.CompilerParams(use_tc_tiling_on_sc=False),
    scratch_types=(
        pltpu.VMEM((NCH, CW), jnp.int32),
        pltpu.VMEM_SHARED((N, H), jnp.float32),
        pltpu.VMEM((CW, H), jnp.float32),
    ),
)
def _sc_degree(dst_hbm, zeros_hbm, ones_hbm, out_hbm, dst_v, acc, ones_v):
    c = lax.axis_index("c")
    s = lax.axis_index("s")
    w = c * NS + s
    r0 = s * RPT

    pltpu.sync_copy(dst_hbm.at[pl.ds(w * NCH, NCH)], dst_v)
    pltpu.sync_copy(ones_hbm, ones_v)
    pltpu.sync_copy(zeros_hbm.at[pl.ds(r0, RPT)], acc.at[pl.ds(r0, RPT)])
    plsc.subcore_barrier()

    def chunk(j, carry):
        pltpu.sync_copy(ones_v, acc.at[dst_v.at[j]], add=True)
        return carry

    lax.fori_loop(0, NCH, chunk, 0)
    plsc.subcore_barrier()
    pltpu.sync_copy(acc.at[pl.ds(r0, RPT)], out_hbm.at[c, pl.ds(r0, RPT)])


def _tc_prologue(xp, W1big, dacc_p):
    def body(x_ref, w_ref, dacc_ref, dinv_ref, g_ref):
        deg = dacc_ref[0] + dacc_ref[1] + 1.0
        dinv = lax.rsqrt(deg)
        dinv_ref[...] = dinv
        h = jnp.dot(x_ref[...], w_ref[...], preferred_element_type=jnp.float32)
        g_ref[...] = dinv * h

    return pl.pallas_call(
        body,
        out_shape=(
            jax.ShapeDtypeStruct((NPK, 128), jnp.float32),
            jax.ShapeDtypeStruct((NPK, 128), jnp.float32),
        ),
    )(xp, W1big, dacc_p)


def _tc_mid(acc_p, g_p, btile, Wbig, dinv_p):
    def body(acc_ref, g_ref, b_ref, w_ref, dinv_ref, out_ref):
        dinv = dinv_ref[...]
        z = dinv * (acc_ref[0] + acc_ref[1] + g_ref[...]) + b_ref[...]
        h = jnp.where(z >= 0.0, z, 0.01 * z)
        out_ref[...] = dinv * jnp.dot(h, w_ref[...],
                                      preferred_element_type=jnp.float32)

    return pl.pallas_call(
        body,
        out_shape=jax.ShapeDtypeStruct((NPK, 128), jnp.float32),
    )(acc_p, g_p, btile, Wbig, dinv_p)


def _tc_final(acc_p, g_p, btile, dinv_p):
    def body(acc_ref, g_ref, b_ref, dinv_ref, out_ref):
        z = dinv_ref[...] * (acc_ref[0] + acc_ref[1] + g_ref[...]) + b_ref[...]
        z3 = z.reshape(NPK, 8, H)
        z3 = z3 - jnp.max(z3, axis=2, keepdims=True)
        e = jnp.exp(z3)
        out_ref[...] = e / jnp.sum(e, axis=2, keepdims=True)

    return pl.pallas_call(
        body,
        out_shape=jax.ShapeDtypeStruct((NPK, 8, H), jnp.float32),
    )(acc_p, g_p, btile, dinv_p)


def kernel(x, edge_index, W1, b1, W2, b2, W3, b3, W4, b4):
    f32 = jnp.float32
    srcr = edge_index[0].reshape(E // CW, CW)
    dstr = edge_index[1].reshape(E // CW, CW)
    xp = x.reshape(NPK, 8 * F_IN)

    eye8 = jnp.eye(8, dtype=f32)
    zeros = jnp.zeros((N, H), f32)
    ones_rows = jnp.ones((CW, H), f32)

    dacc = _sc_degree(dstr, zeros, ones_rows)
    dinv_p, g_p = _tc_prologue(xp, jnp.kron(eye8, W1), dacc.reshape(NC, NPK, 128))

    for b, Wn in ((b1, W2), (b2, W3), (b3, W4)):
        acc = _sc_edge_aggregate(g_p.reshape(N, H), srcr, dstr, zeros)
        g_p = _tc_mid(acc.reshape(NC, NPK, 128), g_p,
                      jnp.tile(b, 8).reshape(1, 128), jnp.kron(eye8, Wn), dinv_p)

    acc = _sc_edge_aggregate(g_p.reshape(N, H), srcr, dstr, zeros)
    out = _tc_final(acc.reshape(NC, NPK, 128), g_p,
                    jnp.tile(b4, 8).reshape(1, 128), dinv_p)
    return out.reshape(N, H)
